# Initial kernel scaffold; baseline (speedup 1.0000x reference)
import jax
import jax.numpy as jnp
from jax.experimental import pallas as pl


def kernel(x, edge_index, batch, cond, params):
    raise NotImplementedError("write your pallas kernel here")


# speedup vs baseline: 5.5594x; 5.5594x over previous
import functools

import jax
import jax.numpy as jnp
from jax import lax
from jax.experimental import pallas as pl
from jax.experimental.pallas import tpu as pltpu
from jax.experimental.pallas import tpu_sc as plsc

N = 10000
E = 320000
D = 128
G = 64
EPS = 1e-5

GROUP = 128
NGROUPS = E // GROUP
NSUB = 16
NCORE = 2
NW = NSUB * NCORE
BASE_G = NGROUPS // NW
EXTRA_G = NGROUPS - BASE_G * NW
N_PAD = 10240
RPS = N_PAD // NSUB

BN_ROWS = 1000
NBLK = N // BN_ROWS


def _sc_scatter_body(h_hbm, ei_hbm, z_hbm, out_hbm, idx_s, idx_d, rows_v,
                     accum, gsem):
    c = lax.axis_index("c")
    s = lax.axis_index("s")
    w = s * NCORE + c

    pltpu.sync_copy(z_hbm, accum.at[pl.ds(s * RPS, RPS)])
    plsc.subcore_barrier()

    cnt = BASE_G + jnp.where(w < EXTRA_G, 1, 0)
    start = w * BASE_G + jnp.minimum(w, EXTRA_G)

    def body(i, carry):
        g = start + i
        pltpu.sync_copy(ei_hbm.at[0, pl.ds(g * GROUP, GROUP)], idx_s)
        pltpu.sync_copy(ei_hbm.at[1, pl.ds(g * GROUP, GROUP)], idx_d)
        pltpu.async_copy(h_hbm.at[idx_s], rows_v, gsem).wait()
        pltpu.sync_copy(rows_v, accum.at[idx_d], add=True)
        return carry

    lax.fori_loop(0, cnt, body, 0)
    plsc.subcore_barrier()

    pltpu.sync_copy(accum.at[pl.ds(s * RPS, RPS)],
                    out_hbm.at[c, pl.ds(s * RPS, RPS)])


@jax.jit
def _sc_scatter(h, edge_index, zrows):
    mesh = plsc.VectorSubcoreMesh(core_axis_name="c", subcore_axis_name="s")
    f = pl.kernel(
        _sc_scatter_body,
        out_type=jax.ShapeDtypeStruct((NCORE, N_PAD, D), jnp.float32),
        mesh=mesh,
        scratch_types=[
            pltpu.VMEM((GROUP,), jnp.int32),
            pltpu.VMEM((GROUP,), jnp.int32),
            pltpu.VMEM((GROUP, D), jnp.float32),
            pltpu.VMEM_SHARED((N_PAD, D), jnp.float32),
            pltpu.SemaphoreType.DMA,
        ],
    )
    return f(h, edge_index, zrows)


def _leaky(t):
    return jnp.where(t >= 0, t, 0.2 * t)


def _stage_a_body(x_ref, a_ref, w1_ref, b1_ref, t_ref, st_ref):
    u = x_ref[...] + a_ref[0] + a_ref[1]
    t = _leaky(jnp.dot(u, w1_ref[...], preferred_element_type=jnp.float32)
               + b1_ref[...])
    t_ref[...] = t

    @pl.when(pl.program_id(0) == 0)
    def _():
        st_ref[...] = jnp.zeros_like(st_ref)

    s1 = jnp.sum(t, axis=0, keepdims=True)
    s2 = jnp.sum(t * t, axis=0, keepdims=True)
    st_ref[...] += jnp.concatenate(
        [s1, s2, jnp.zeros((6, D), jnp.float32)], axis=0)


def _stage_a(h, aggr, w1, b1):
    return pl.pallas_call(
        _stage_a_body,
        grid=(NBLK,),
        in_specs=[
            pl.BlockSpec((BN_ROWS, D), lambda i: (i, 0)),
            pl.BlockSpec((NCORE, BN_ROWS, D), lambda i: (0, i, 0)),
            pl.BlockSpec((D, D), lambda i: (0, 0)),
            pl.BlockSpec((1, D), lambda i: (0, 0)),
        ],
        out_specs=[
            pl.BlockSpec((BN_ROWS, D), lambda i: (i, 0)),
            pl.BlockSpec((8, D), lambda i: (0, 0)),
        ],
        out_shape=[
            jax.ShapeDtypeStruct((N, D), jnp.float32),
            jax.ShapeDtypeStruct((8, D), jnp.float32),
        ],
    )(h, aggr, w1, b1)


def _stage_b_body(t_ref, st_ref, g_ref, b_ref, w2_ref, b2_ref, h_ref):
    mu = st_ref[0:1, :] * (1.0 / N)
    var = st_ref[1:2, :] * (1.0 / N) - mu * mu
    y = (t_ref[...] - mu) / jnp.sqrt(var + EPS) * g_ref[...] + b_ref[...]
    z = jnp.dot(y, w2_ref[...], preferred_element_type=jnp.float32) + b2_ref[...]
    h_ref[...] = _leaky(z)


def _stage_b(t, st, g, b, w2, b2):
    return pl.pallas_call(
        _stage_b_body,
        grid=(NBLK,),
        in_specs=[
            pl.BlockSpec((BN_ROWS, D), lambda i: (i, 0)),
            pl.BlockSpec((8, D), lambda i: (0, 0)),
            pl.BlockSpec((1, D), lambda i: (0, 0)),
            pl.BlockSpec((1, D), lambda i: (0, 0)),
            pl.BlockSpec((D, D), lambda i: (0, 0)),
            pl.BlockSpec((1, D), lambda i: (0, 0)),
        ],
        out_specs=pl.BlockSpec((BN_ROWS, D), lambda i: (i, 0)),
        out_shape=jax.ShapeDtypeStruct((N, D), jnp.float32),
    )(t, st, g, b, w2, b2)


def _stage_b3_body(t_ref, st_ref, g_ref, b_ref, w2_ref, b2_ref, batch_ref,
                   pooled_ref):
    mu = st_ref[0:1, :] * (1.0 / N)
    var = st_ref[1:2, :] * (1.0 / N) - mu * mu
    y = (t_ref[...] - mu) / jnp.sqrt(var + EPS) * g_ref[...] + b_ref[...]
    z = jnp.dot(y, w2_ref[...], preferred_element_type=jnp.float32) + b2_ref[...]
    h = _leaky(z)

    bt = batch_ref[0, 0, :]
    oh = (bt[None, :] == lax.broadcasted_iota(jnp.int32, (G, BN_ROWS), 0)
          ).astype(jnp.float32)

    @pl.when(pl.program_id(0) == 0)
    def _():
        pooled_ref[...] = jnp.zeros_like(pooled_ref)

    pooled_ref[...] += jnp.dot(oh, h, preferred_element_type=jnp.float32)


def _stage_b3(t, st, g, b, w2, b2, batch3):
    return pl.pallas_call(
        _stage_b3_body,
        grid=(NBLK,),
        in_specs=[
            pl.BlockSpec((BN_ROWS, D), lambda i: (i, 0)),
            pl.BlockSpec((8, D), lambda i: (0, 0)),
            pl.BlockSpec((1, D), lambda i: (0, 0)),
            pl.BlockSpec((1, D), lambda i: (0, 0)),
            pl.BlockSpec((D, D), lambda i: (0, 0)),
            pl.BlockSpec((1, D), lambda i: (0, 0)),
            pl.BlockSpec((1, 1, BN_ROWS), lambda i: (i, 0, 0)),
        ],
        out_specs=pl.BlockSpec((G, D), lambda i: (0, 0)),
        out_shape=jax.ShapeDtypeStruct((G, D), jnp.float32),
    )(t, st, g, b, w2, b2, batch3)


def _final_body(condp_ref, cw1_ref, cb1_ref, cg_ref, cb_ref, cw2_ref, cb2_ref,
                pooled_ref, gc_ref, bc_ref, gp_ref, bp_ref, fwc_ref, fwp_ref,
                fb_ref, out_ref):
    c = jnp.dot(condp_ref[...], cw1_ref[...],
                preferred_element_type=jnp.float32) + cb1_ref[...]
    mu = jnp.mean(c, axis=0, keepdims=True)
    var = jnp.mean((c - mu) ** 2, axis=0, keepdims=True)
    c = (c - mu) / jnp.sqrt(var + EPS) * cg_ref[...] + cb_ref[...]
    c = jnp.maximum(c, 0.0)
    c = jnp.dot(c, cw2_ref[...], preferred_element_type=jnp.float32) + cb2_ref[...]
    c = jnp.maximum(c, 0.0)

    muc = jnp.mean(c, axis=0, keepdims=True)
    varc = jnp.mean((c - muc) ** 2, axis=0, keepdims=True)
    cn = (c - muc) / jnp.sqrt(varc + EPS) * gc_ref[...] + bc_ref[...]

    p = pooled_ref[...]
    mup = jnp.mean(p, axis=0, keepdims=True)
    varp = jnp.mean((p - mup) ** 2, axis=0, keepdims=True)
    pn = (p - mup) / jnp.sqrt(varp + EPS) * gp_ref[...] + bp_ref[...]

    out_ref[...] = (jnp.dot(cn, fwc_ref[...], preferred_element_type=jnp.float32)
                    + jnp.dot(pn, fwp_ref[...], preferred_element_type=jnp.float32)
                    + fb_ref[...])


def _final(condp, cw1, cb1, cg, cb, cw2, cb2, pooled, gc, bc, gp, bp, fwc,
           fwp, fb):
    return pl.pallas_call(
        _final_body,
        out_shape=jax.ShapeDtypeStruct((G, 64), jnp.float32),
    )(condp, cw1, cb1, cg, cb, cw2, cb2, pooled, gc, bc, gp, bp, fwc, fwp, fb)


def kernel(x, edge_index, batch, cond, params):
    zrows = jnp.zeros((RPS, D), jnp.float32)
    batch3 = batch.reshape(NBLK, 1, BN_ROWS)

    h = x
    for i, p in enumerate(params['convs']):
        aggr = _sc_scatter(h, edge_index, zrows)
        t, st = _stage_a(h, aggr, p['W1'], p['b1'].reshape(1, D))
        if i < 2:
            h = _stage_b(t, st, p['bn_g'].reshape(1, D),
                         p['bn_b'].reshape(1, D), p['W2'],
                         p['b2'].reshape(1, D))
        else:
            pooled = _stage_b3(t, st, p['bn_g'].reshape(1, D),
                               p['bn_b'].reshape(1, D), p['W2'],
                               p['b2'].reshape(1, D), batch3)

    CD, CH = 7, 5
    condp = jnp.zeros((G, D), jnp.float32).at[:, :CD].set(cond)
    cw1 = jnp.zeros((D, D), jnp.float32).at[:CD, :CH].set(params['cond_W1'])
    cb1 = jnp.zeros((1, D), jnp.float32).at[0, :CH].set(params['cond_b1'])
    cg = jnp.zeros((1, D), jnp.float32).at[0, :CH].set(params['cond_bn_g'])
    cb = jnp.zeros((1, D), jnp.float32).at[0, :CH].set(params['cond_bn_b'])
    cw2 = jnp.zeros((D, D), jnp.float32).at[:CH, :CH].set(params['cond_W2'])
    cb2 = jnp.zeros((1, D), jnp.float32).at[0, :CH].set(params['cond_b2'])
    gc = jnp.zeros((1, D), jnp.float32).at[0, :CH].set(params['bn_g'][:CH])
    bc = jnp.zeros((1, D), jnp.float32).at[0, :CH].set(params['bn_b'][:CH])
    gp = params['bn_g'][CH:].reshape(1, D)
    bp = params['bn_b'][CH:].reshape(1, D)
    fwc = jnp.zeros((D, 64), jnp.float32).at[:CH, :].set(params['fc_W'][:CH])
    fwp = params['fc_W'][CH:]
    fb = params['fc_b'].reshape(1, 64)

    return _final(condp, cw1, cb1, cg, cb, cw2, cb2, pooled, gc, bc, gp, bp,
                  fwc, fwp, fb)


# speedup vs baseline: 8.5569x; 1.5392x over previous
import functools

import jax
import jax.numpy as jnp
from jax import lax
from jax.experimental import pallas as pl
from jax.experimental.pallas import tpu as pltpu
from jax.experimental.pallas import tpu_sc as plsc

N = 10000
E = 320000
D = 128
G = 64
EPS = 1e-5

GROUP = 128
NGROUPS = E // GROUP
NPAIRS = NGROUPS // 2
NSUB = 16
NCORE = 2
NW = NSUB * NCORE
BASE_P = NPAIRS // NW
EXTRA_P = NPAIRS - BASE_P * NW
N_PAD = 10240
RPS = N_PAD // NSUB

BN_ROWS = 1000
NBLK = N // BN_ROWS


def _sc_scatter_body(h_hbm, ei_hbm, z_hbm, out_hbm, idx_s, idx_d, rows_v,
                     accum, sem0, sem1):
    c = lax.axis_index("c")
    s = lax.axis_index("s")
    w = s * NCORE + c

    pltpu.sync_copy(z_hbm, accum.at[pl.ds(s * RPS, RPS)])
    plsc.subcore_barrier()

    cntp = BASE_P + jnp.where(w < EXTRA_P, 1, 0)
    startp = w * BASE_P + jnp.minimum(w, EXTRA_P)

    def load_idx(g, b):
        pltpu.sync_copy(ei_hbm.at[0, pl.ds(g * GROUP, GROUP)], idx_s.at[b])
        pltpu.sync_copy(ei_hbm.at[1, pl.ds(g * GROUP, GROUP)], idx_d.at[b])

    g_first = 2 * startp
    load_idx(g_first, 0)
    pltpu.async_copy(h_hbm.at[idx_s.at[0]], rows_v.at[0], sem0)

    def body(j, carry):
        g0 = 2 * (startp + j)
        load_idx(g0 + 1, 1)
        cp1 = pltpu.async_copy(h_hbm.at[idx_s.at[1]], rows_v.at[1], sem1)
        pltpu.make_async_copy(h_hbm.at[pl.ds(0, GROUP)], rows_v.at[0],
                              sem0).wait()
        pltpu.sync_copy(rows_v.at[0], accum.at[idx_d.at[0]], add=True)

        @pl.when(j + 1 < cntp)
        def _():
            load_idx(g0 + 2, 0)
            pltpu.async_copy(h_hbm.at[idx_s.at[0]], rows_v.at[0], sem0)

        cp1.wait()
        pltpu.sync_copy(rows_v.at[1], accum.at[idx_d.at[1]], add=True)
        return carry

    lax.fori_loop(0, cntp, body, 0)
    plsc.subcore_barrier()

    pltpu.sync_copy(accum.at[pl.ds(s * RPS, RPS)],
                    out_hbm.at[c, pl.ds(s * RPS, RPS)])


@jax.jit
def _sc_scatter(h, edge_index, zrows):
    mesh = plsc.VectorSubcoreMesh(core_axis_name="c", subcore_axis_name="s")
    f = pl.kernel(
        _sc_scatter_body,
        out_type=jax.ShapeDtypeStruct((NCORE, N_PAD, D), jnp.float32),
        mesh=mesh,
        scratch_types=[
            pltpu.VMEM((2, GROUP), jnp.int32),
            pltpu.VMEM((2, GROUP), jnp.int32),
            pltpu.VMEM((2, GROUP, D), jnp.float32),
            pltpu.VMEM_SHARED((N_PAD, D), jnp.float32),
            pltpu.SemaphoreType.DMA,
            pltpu.SemaphoreType.DMA,
        ],
    )
    return f(h, edge_index, zrows)


def _leaky(t):
    return jnp.where(t >= 0, t, 0.2 * t)


def _stage_a_body(x_ref, a_ref, w1_ref, b1_ref, t_ref, st_ref):
    u = x_ref[...] + a_ref[0] + a_ref[1]
    t = _leaky(jnp.dot(u, w1_ref[...], preferred_element_type=jnp.float32)
               + b1_ref[...])
    t_ref[...] = t

    @pl.when(pl.program_id(0) == 0)
    def _():
        st_ref[...] = jnp.zeros_like(st_ref)

    s1 = jnp.sum(t, axis=0, keepdims=True)
    s2 = jnp.sum(t * t, axis=0, keepdims=True)
    st_ref[...] += jnp.concatenate(
        [s1, s2, jnp.zeros((6, D), jnp.float32)], axis=0)


def _stage_a(h, aggr, w1, b1):
    return pl.pallas_call(
        _stage_a_body,
        grid=(NBLK,),
        in_specs=[
            pl.BlockSpec((BN_ROWS, D), lambda i: (i, 0)),
            pl.BlockSpec((NCORE, BN_ROWS, D), lambda i: (0, i, 0)),
            pl.BlockSpec((D, D), lambda i: (0, 0)),
            pl.BlockSpec((1, D), lambda i: (0, 0)),
        ],
        out_specs=[
            pl.BlockSpec((BN_ROWS, D), lambda i: (i, 0)),
            pl.BlockSpec((8, D), lambda i: (0, 0)),
        ],
        out_shape=[
            jax.ShapeDtypeStruct((N, D), jnp.float32),
            jax.ShapeDtypeStruct((8, D), jnp.float32),
        ],
    )(h, aggr, w1, b1)


def _stage_b_body(t_ref, st_ref, g_ref, b_ref, w2_ref, b2_ref, h_ref):
    mu = st_ref[0:1, :] * (1.0 / N)
    var = st_ref[1:2, :] * (1.0 / N) - mu * mu
    y = (t_ref[...] - mu) / jnp.sqrt(var + EPS) * g_ref[...] + b_ref[...]
    z = jnp.dot(y, w2_ref[...], preferred_element_type=jnp.float32) + b2_ref[...]
    h_ref[...] = _leaky(z)


def _stage_b(t, st, g, b, w2, b2):
    return pl.pallas_call(
        _stage_b_body,
        grid=(NBLK,),
        in_specs=[
            pl.BlockSpec((BN_ROWS, D), lambda i: (i, 0)),
            pl.BlockSpec((8, D), lambda i: (0, 0)),
            pl.BlockSpec((1, D), lambda i: (0, 0)),
            pl.BlockSpec((1, D), lambda i: (0, 0)),
            pl.BlockSpec((D, D), lambda i: (0, 0)),
            pl.BlockSpec((1, D), lambda i: (0, 0)),
        ],
        out_specs=pl.BlockSpec((BN_ROWS, D), lambda i: (i, 0)),
        out_shape=jax.ShapeDtypeStruct((N, D), jnp.float32),
    )(t, st, g, b, w2, b2)


def _stage_b3_body(t_ref, st_ref, g_ref, b_ref, w2_ref, b2_ref, batch_ref,
                   pooled_ref):
    mu = st_ref[0:1, :] * (1.0 / N)
    var = st_ref[1:2, :] * (1.0 / N) - mu * mu
    y = (t_ref[...] - mu) / jnp.sqrt(var + EPS) * g_ref[...] + b_ref[...]
    z = jnp.dot(y, w2_ref[...], preferred_element_type=jnp.float32) + b2_ref[...]
    h = _leaky(z)

    bt = batch_ref[0, 0, :]
    oh = (bt[None, :] == lax.broadcasted_iota(jnp.int32, (G, BN_ROWS), 0)
          ).astype(jnp.float32)

    @pl.when(pl.program_id(0) == 0)
    def _():
        pooled_ref[...] = jnp.zeros_like(pooled_ref)

    pooled_ref[...] += jnp.dot(oh, h, preferred_element_type=jnp.float32)


def _stage_b3(t, st, g, b, w2, b2, batch3):
    return pl.pallas_call(
        _stage_b3_body,
        grid=(NBLK,),
        in_specs=[
            pl.BlockSpec((BN_ROWS, D), lambda i: (i, 0)),
            pl.BlockSpec((8, D), lambda i: (0, 0)),
            pl.BlockSpec((1, D), lambda i: (0, 0)),
            pl.BlockSpec((1, D), lambda i: (0, 0)),
            pl.BlockSpec((D, D), lambda i: (0, 0)),
            pl.BlockSpec((1, D), lambda i: (0, 0)),
            pl.BlockSpec((1, 1, BN_ROWS), lambda i: (i, 0, 0)),
        ],
        out_specs=pl.BlockSpec((G, D), lambda i: (0, 0)),
        out_shape=jax.ShapeDtypeStruct((G, D), jnp.float32),
    )(t, st, g, b, w2, b2, batch3)


def _final_body(condp_ref, cw1_ref, cb1_ref, cg_ref, cb_ref, cw2_ref, cb2_ref,
                pooled_ref, gc_ref, bc_ref, gp_ref, bp_ref, fwc_ref, fwp_ref,
                fb_ref, out_ref):
    c = jnp.dot(condp_ref[...], cw1_ref[...],
                preferred_element_type=jnp.float32) + cb1_ref[...]
    mu = jnp.mean(c, axis=0, keepdims=True)
    var = jnp.mean((c - mu) ** 2, axis=0, keepdims=True)
    c = (c - mu) / jnp.sqrt(var + EPS) * cg_ref[...] + cb_ref[...]
    c = jnp.maximum(c, 0.0)
    c = jnp.dot(c, cw2_ref[...], preferred_element_type=jnp.float32) + cb2_ref[...]
    c = jnp.maximum(c, 0.0)

    muc = jnp.mean(c, axis=0, keepdims=True)
    varc = jnp.mean((c - muc) ** 2, axis=0, keepdims=True)
    cn = (c - muc) / jnp.sqrt(varc + EPS) * gc_ref[...] + bc_ref[...]

    p = pooled_ref[...]
    mup = jnp.mean(p, axis=0, keepdims=True)
    varp = jnp.mean((p - mup) ** 2, axis=0, keepdims=True)
    pn = (p - mup) / jnp.sqrt(varp + EPS) * gp_ref[...] + bp_ref[...]

    out_ref[...] = (jnp.dot(cn, fwc_ref[...], preferred_element_type=jnp.float32)
                    + jnp.dot(pn, fwp_ref[...], preferred_element_type=jnp.float32)
                    + fb_ref[...])


def _final(condp, cw1, cb1, cg, cb, cw2, cb2, pooled, gc, bc, gp, bp, fwc,
           fwp, fb):
    return pl.pallas_call(
        _final_body,
        out_shape=jax.ShapeDtypeStruct((G, 64), jnp.float32),
    )(condp, cw1, cb1, cg, cb, cw2, cb2, pooled, gc, bc, gp, bp, fwc, fwp, fb)


def kernel(x, edge_index, batch, cond, params):
    zrows = jnp.zeros((RPS, D), jnp.float32)
    batch3 = batch.reshape(NBLK, 1, BN_ROWS)

    h = x
    for i, p in enumerate(params['convs']):
        aggr = _sc_scatter(h, edge_index, zrows)
        t, st = _stage_a(h, aggr, p['W1'], p['b1'].reshape(1, D))
        if i < 2:
            h = _stage_b(t, st, p['bn_g'].reshape(1, D),
                         p['bn_b'].reshape(1, D), p['W2'],
                         p['b2'].reshape(1, D))
        else:
            pooled = _stage_b3(t, st, p['bn_g'].reshape(1, D),
                               p['bn_b'].reshape(1, D), p['W2'],
                               p['b2'].reshape(1, D), batch3)

    CD, CH = 7, 5
    condp = jnp.zeros((G, D), jnp.float32).at[:, :CD].set(cond)
    cw1 = jnp.zeros((D, D), jnp.float32).at[:CD, :CH].set(params['cond_W1'])
    cb1 = jnp.zeros((1, D), jnp.float32).at[0, :CH].set(params['cond_b1'])
    cg = jnp.zeros((1, D), jnp.float32).at[0, :CH].set(params['cond_bn_g'])
    cb = jnp.zeros((1, D), jnp.float32).at[0, :CH].set(params['cond_bn_b'])
    cw2 = jnp.zeros((D, D), jnp.float32).at[:CH, :CH].set(params['cond_W2'])
    cb2 = jnp.zeros((1, D), jnp.float32).at[0, :CH].set(params['cond_b2'])
    gc = jnp.zeros((1, D), jnp.float32).at[0, :CH].set(params['bn_g'][:CH])
    bc = jnp.zeros((1, D), jnp.float32).at[0, :CH].set(params['bn_b'][:CH])
    gp = params['bn_g'][CH:].reshape(1, D)
    bp = params['bn_b'][CH:].reshape(1, D)
    fwc = jnp.zeros((D, 64), jnp.float32).at[:CH, :].set(params['fc_W'][:CH])
    fwp = params['fc_W'][CH:]
    fb = params['fc_b'].reshape(1, 64)

    return _final(condp, cw1, cb1, cg, cb, cw2, cb2, pooled, gc, bc, gp, bp,
                  fwc, fwp, fb)
